# baseline (speedup 1.0000x reference)
import functools

import jax
import jax.numpy as jnp
from jax import lax
from jax.experimental import pallas as pl
from jax.experimental.pallas import tpu as pltpu
from jax.experimental.pallas import tpu_sc as plsc

N = 10000
E = 320000
NPAD = 10240
IN_DIM = 128
HID = 128
OUT_PAD = 64

R = 512
GRID = NPAD // R

CH = 80
WORKERS = 32
EPT = E // WORKERS
NCH = EPT // CH
G = CH // 16
NPT = N // 16


def _mm_body(x_ref, w_ref, as_ref, ad_ref, h_ref, es_ref, ed_ref):
    h = jnp.dot(x_ref[...], w_ref[...], preferred_element_type=jnp.float32)
    h_ref[...] = h
    es_ref[...] = jnp.dot(h, as_ref[...], preferred_element_type=jnp.float32)
    ed_ref[...] = jnp.dot(h, ad_ref[...], preferred_element_type=jnp.float32)


def _combine_mm_body(n0_ref, n1_ref, d0_ref, d1_ref, w_ref, as_ref, ad_ref,
                     h_ref, es_ref, ed_ref):
    den = d0_ref[...] + d1_ref[...] + 1e-16
    hin = (n0_ref[...] + n1_ref[...]) / den
    h = jnp.dot(hin, w_ref[...], preferred_element_type=jnp.float32)
    h_ref[...] = h
    es_ref[...] = jnp.dot(h, as_ref[...], preferred_element_type=jnp.float32)
    ed_ref[...] = jnp.dot(h, ad_ref[...], preferred_element_type=jnp.float32)


def _lsm_body(n0_ref, n1_ref, d0_ref, d1_ref, o_ref):
    den = d0_ref[...] + d1_ref[...] + 1e-16
    logits = (n0_ref[...] + n1_ref[...]) / den
    col = lax.broadcasted_iota(jnp.int32, logits.shape, 1)
    ml = jnp.where(col < 40, logits, -1e30)
    m = jnp.max(ml, axis=1, keepdims=True)
    l = ml - m
    s = jnp.sum(jnp.exp(l), axis=1, keepdims=True)
    o_ref[...] = l - jnp.log(s)


def _row_spec(d):
    return pl.BlockSpec((R, d), lambda i: (i, 0))


def _full_spec(a, b):
    return pl.BlockSpec((a, b), lambda i: (0, 0))


def _make_prep(d_in, d_out, combine):
    body = _combine_mm_body if combine else _mm_body
    in_specs = []
    if combine:
        in_specs += [_row_spec(d_in), _row_spec(d_in),
                     _row_spec(1), _row_spec(1)]
    else:
        in_specs += [_row_spec(d_in)]
    in_specs += [_full_spec(d_in, d_out), _full_spec(d_out, 1),
                 _full_spec(d_out, 1)]
    return pl.pallas_call(
        body,
        grid=(GRID,),
        in_specs=in_specs,
        out_specs=[_row_spec(d_out), _row_spec(1), _row_spec(1)],
        out_shape=[
            jax.ShapeDtypeStruct((NPAD, d_out), jnp.float32),
            jax.ShapeDtypeStruct((NPAD, 1), jnp.float32),
            jax.ShapeDtypeStruct((NPAD, 1), jnp.float32),
        ],
    )


_prep1 = _make_prep(IN_DIM, HID, combine=False)
_prep2 = _make_prep(HID, HID, combine=True)
_prep3 = _make_prep(HID, OUT_PAD, combine=True)

_final = pl.pallas_call(
    _lsm_body,
    grid=(GRID,),
    in_specs=[_row_spec(64), _row_spec(64), _row_spec(1), _row_spec(1)],
    out_specs=_row_spec(64),
    out_shape=jax.ShapeDtypeStruct((NPAD, 64), jnp.float32),
)


def _make_edge_kernel(D):
    mesh = plsc.VectorSubcoreMesh(core_axis_name="c", subcore_axis_name="s")

    @functools.partial(
        pl.kernel,
        out_type=[
            jax.ShapeDtypeStruct((2, N, D), jnp.float32),
            jax.ShapeDtypeStruct((2, NPAD), jnp.float32),
        ],
        mesh=mesh,
        compiler_params=pltpu.CompilerParams(
            needs_layout_passes=False, use_tc_tiling_on_sc=False),
        scratch_types=[
            pltpu.VMEM((3, CH), jnp.int32),
            pltpu.VMEM((3, CH), jnp.int32),
            pltpu.VMEM((3, CH), jnp.int32),
            pltpu.VMEM((3, CH), jnp.float32),
            pltpu.VMEM((3, CH), jnp.float32),
            pltpu.VMEM((3, CH), jnp.float32),
            pltpu.VMEM((3, CH, D), jnp.float32),
            pltpu.VMEM((RPT := NPAD // 16,), jnp.float32),
            pltpu.VMEM_SHARED((16, EPT), jnp.int32),
            pltpu.VMEM_SHARED((NPAD,), jnp.float32),
            pltpu.VMEM_SHARED((NPAD,), jnp.float32),
            pltpu.VMEM_SHARED((N, D), jnp.float32),
            pltpu.VMEM_SHARED((NPAD,), jnp.float32),
            pltpu.SemaphoreType.DMA,
            pltpu.SemaphoreType.DMA,
            pltpu.SemaphoreType.DMA,
            pltpu.SemaphoreType.DMA,
            pltpu.SemaphoreType.DMA,
            pltpu.SemaphoreType.DMA,
            pltpu.SemaphoreType.DMA,
            pltpu.SemaphoreType.DMA,
            pltpu.SemaphoreType.DMA,
        ],
    )
    def edge_kernel(h_hbm, es_hbm, ed_hbm, pk_hbm,
                    num_out, den_out,
                    pk_v, src_v, dst_v, esb_v, edb_v, w_v, rows_v, zden_v,
                    edge_sh, es_sh, ed_sh, num_acc, den_acc,
                    es0, es1, es2, gs0, gs1, gs2, ss0, ss1, ss2):
        cid = lax.axis_index("c")
        sid = lax.axis_index("s")
        wid = sid * 2 + cid
        esems = (es0, es1, es2)
        gsems = (gs0, gs1, gs2)
        ssems = (ss0, ss1, ss2)
        zero16 = jnp.zeros((16,), jnp.float32)
        RPTC = NPAD // 16

        def zrow(i, _):
            for c8 in range(D // 16):
                rows_v[0, i, pl.ds(c8 * 16, 16)] = zero16
            return 0
        lax.fori_loop(0, CH, zrow, 0)

        def zden(i, _):
            zden_v[pl.ds(i * 16, 16)] = zero16
            return 0
        lax.fori_loop(0, RPTC // 16, zden, 0)

        base_n = sid * NPT
        base_r = sid * RPTC
        for r in range(7):
            pltpu.sync_copy(rows_v.at[0],
                            num_acc.at[pl.ds(base_n + r * CH, CH)])
        pltpu.sync_copy(rows_v.at[0, pl.ds(0, NPT - 7 * CH)],
                        num_acc.at[pl.ds(base_n + 7 * CH, NPT - 7 * CH)])
        pltpu.sync_copy(zden_v, den_acc.at[pl.ds(base_r, RPTC)])

        pltpu.sync_copy(es_hbm.at[pl.ds(base_r, RPTC)],
                        es_sh.at[pl.ds(base_r, RPTC)])
        pltpu.sync_copy(ed_hbm.at[pl.ds(base_r, RPTC)],
                        ed_sh.at[pl.ds(base_r, RPTC)])
        pltpu.sync_copy(pk_hbm.at[pl.ds(wid * EPT, EPT)], edge_sh.at[sid])
        plsc.subcore_barrier()

        def prep(t, b):
            pltpu.sync_copy(edge_sh.at[sid, pl.ds(t * CH, CH)], pk_v.at[b])
            for g in range(G):
                sl = pl.ds(g * 16, 16)
                p16 = pk_v[b, sl]
                src_v[b, sl] = jnp.bitwise_and(p16, 16383)
                dst_v[b, sl] = jnp.right_shift(p16, 14)
            pltpu.async_copy(es_sh.at[src_v.at[b]], esb_v.at[b], esems[b])
            pltpu.async_copy(ed_sh.at[dst_v.at[b]], edb_v.at[b], esems[b])
            pltpu.async_copy(h_hbm.at[src_v.at[b]], rows_v.at[b], gsems[b])

        def wait_esed(b):
            pltpu.make_async_copy(es_sh.at[src_v.at[b]], esb_v.at[b],
                                  esems[b]).wait()
            pltpu.make_async_copy(ed_sh.at[dst_v.at[b]], edb_v.at[b],
                                  esems[b]).wait()

        def wait_gather(b):
            pltpu.make_async_copy(h_hbm.at[src_v.at[b]], rows_v.at[b],
                                  gsems[b]).wait()

        def start_scatter(b):
            pltpu.async_copy(rows_v.at[b], num_acc.at[dst_v.at[b]],
                             ssems[b], add=True)
            pltpu.async_copy(w_v.at[b], den_acc.at[dst_v.at[b]],
                             ssems[b], add=True)

        def wait_scatter(b):
            pltpu.make_async_copy(rows_v.at[b], num_acc.at[dst_v.at[b]],
                                  ssems[b]).wait()
            pltpu.make_async_copy(w_v.at[b], den_acc.at[dst_v.at[b]],
                                  ssems[b]).wait()

        def compute_w(b):
            wait_esed(b)
            for g in range(G):
                sl = pl.ds(g * 16, 16)
                e = esb_v[b, sl] + edb_v[b, sl]
                e = jnp.where(e > 0, e, e * jnp.float32(0.2))
                e = jnp.minimum(e, jnp.float32(70.0))
                w_v[b, sl] = jnp.exp(e)

        def scale(b):
            def srow16(g, _):
                w16 = w_v[b, pl.ds(g * 16, 16)]
                for j in range(16):
                    wb = jnp.broadcast_to(w16[j], (16,))
                    i = g * 16 + j
                    for c8 in range(D // 16):
                        sl = pl.ds(c8 * 16, 16)
                        rows_v[b, i, sl] = rows_v[b, i, sl] * wb
                return 0
            lax.fori_loop(0, G, srow16, 0)

        def body(t, b):
            compute_w(b)
            b1 = (b + 1) % 3

            @pl.when(t >= 2)
            def _():
                wait_scatter(b1)

            @pl.when(t <= NCH - 2)
            def _():
                prep(t + 1, b1)
            wait_gather(b)
            scale(b)
            start_scatter(b)

        prep(0, 0)

        def uloop(u, _):
            for k in range(3):
                t = 3 * u + k

                @pl.when(t <= NCH - 1)
                def _():
                    body(t, k)
            return 0
        lax.fori_loop(0, (NCH + 2) // 3, uloop, 0)
        wait_scatter((NCH - 2) % 3)
        wait_scatter((NCH - 1) % 3)

        plsc.subcore_barrier()
        pltpu.sync_copy(num_acc.at[pl.ds(base_n, NPT)],
                        num_out.at[cid, pl.ds(base_n, NPT)])
        pltpu.sync_copy(den_acc.at[pl.ds(base_r, RPTC)],
                        den_out.at[cid, pl.ds(base_r, RPTC)])

    return edge_kernel


_edge128 = _make_edge_kernel(HID)
_edge64 = _make_edge_kernel(64)


def kernel(x, edge_index, W1, a_src1, a_dst1, W2, a_src2, a_dst2,
           W3, a_src3, a_dst3):
    ei = edge_index.astype(jnp.int32)
    pk = jnp.bitwise_or(ei[0], jnp.left_shift(ei[1], 14))

    W3p = jnp.zeros((HID, OUT_PAD), jnp.float32).at[:, :40].set(W3)
    a_s3 = jnp.zeros((OUT_PAD,), jnp.float32).at[:40].set(a_src3)
    a_d3 = jnp.zeros((OUT_PAD,), jnp.float32).at[:40].set(a_dst3)

    h1, es1, ed1 = _prep1(x, W1, a_src1.reshape(HID, 1), a_dst1.reshape(HID, 1))
    num1, den1 = _edge128(h1, es1.reshape(NPAD), ed1.reshape(NPAD), pk)

    h2, es2, ed2 = _prep2(num1[0], num1[1],
                          den1[0].reshape(NPAD, 1), den1[1].reshape(NPAD, 1),
                          W2, a_src2.reshape(HID, 1), a_dst2.reshape(HID, 1))
    num2, den2 = _edge128(h2, es2.reshape(NPAD), ed2.reshape(NPAD), pk)

    h3, es3, ed3 = _prep3(num2[0], num2[1],
                          den2[0].reshape(NPAD, 1), den2[1].reshape(NPAD, 1),
                          W3p, a_s3.reshape(OUT_PAD, 1), a_d3.reshape(OUT_PAD, 1))
    num3, den3 = _edge64(h3, es3.reshape(NPAD), ed3.reshape(NPAD), pk)

    out = _final(num3[0], num3[1],
                 den3[0].reshape(NPAD, 1), den3[1].reshape(NPAD, 1))
    return out[:N, :40]


# speedup vs baseline: 1.1360x; 1.1360x over previous
import functools

import jax
import jax.numpy as jnp
from jax import lax
from jax.experimental import pallas as pl
from jax.experimental.pallas import tpu as pltpu
from jax.experimental.pallas import tpu_sc as plsc

N = 10000
E = 320000
NPAD = 10240
IN_DIM = 128
HID = 128
OUT_PAD = 128

R = 512
GRID = NPAD // R

CH = 80
WORKERS = 32
EPT = E // WORKERS
NCH = EPT // CH
G = CH // 16
NPT = N // 16


def _mm_body(x_ref, w_ref, as_ref, ad_ref, h_ref, es_ref, ed_ref):
    h = jnp.dot(x_ref[...], w_ref[...], preferred_element_type=jnp.float32)
    h_ref[...] = h
    es_ref[...] = jnp.dot(h, as_ref[...], preferred_element_type=jnp.float32)
    ed_ref[...] = jnp.dot(h, ad_ref[...], preferred_element_type=jnp.float32)


def _combine_mm_body(n0_ref, n1_ref, d0_ref, d1_ref, w_ref, as_ref, ad_ref,
                     h_ref, es_ref, ed_ref):
    den = d0_ref[...] + d1_ref[...] + 1e-16
    hin = (n0_ref[...] + n1_ref[...]) / den
    h = jnp.dot(hin, w_ref[...], preferred_element_type=jnp.float32)
    h_ref[...] = h
    es_ref[...] = jnp.dot(h, as_ref[...], preferred_element_type=jnp.float32)
    ed_ref[...] = jnp.dot(h, ad_ref[...], preferred_element_type=jnp.float32)


def _lsm_body(n0_ref, n1_ref, d0_ref, d1_ref, o_ref):
    den = d0_ref[...] + d1_ref[...] + 1e-16
    logits = (n0_ref[...] + n1_ref[...]) / den
    col = lax.broadcasted_iota(jnp.int32, logits.shape, 1)
    ml = jnp.where(col < 40, logits, -1e30)
    m = jnp.max(ml, axis=1, keepdims=True)
    l = ml - m
    s = jnp.sum(jnp.exp(l), axis=1, keepdims=True)
    o_ref[...] = l - jnp.log(s)


def _row_spec(d):
    return pl.BlockSpec((R, d), lambda i: (i, 0))


def _full_spec(a, b):
    return pl.BlockSpec((a, b), lambda i: (0, 0))


def _make_prep(d_in, d_out, combine):
    body = _combine_mm_body if combine else _mm_body
    in_specs = []
    if combine:
        in_specs += [_row_spec(d_in), _row_spec(d_in),
                     _row_spec(1), _row_spec(1)]
    else:
        in_specs += [_row_spec(d_in)]
    in_specs += [_full_spec(d_in, d_out), _full_spec(d_out, 1),
                 _full_spec(d_out, 1)]
    return pl.pallas_call(
        body,
        grid=(GRID,),
        in_specs=in_specs,
        out_specs=[_row_spec(d_out), _row_spec(1), _row_spec(1)],
        out_shape=[
            jax.ShapeDtypeStruct((NPAD, d_out), jnp.float32),
            jax.ShapeDtypeStruct((NPAD, 1), jnp.float32),
            jax.ShapeDtypeStruct((NPAD, 1), jnp.float32),
        ],
    )


_prep1 = _make_prep(IN_DIM, HID, combine=False)
_prep2 = _make_prep(HID, HID, combine=True)
_prep3 = _make_prep(HID, OUT_PAD, combine=True)

_final = pl.pallas_call(
    _lsm_body,
    grid=(GRID,),
    in_specs=[_row_spec(OUT_PAD), _row_spec(OUT_PAD), _row_spec(1), _row_spec(1)],
    out_specs=_row_spec(OUT_PAD),
    out_shape=jax.ShapeDtypeStruct((NPAD, OUT_PAD), jnp.float32),
)


def _make_edge_kernel(D):
    mesh = plsc.VectorSubcoreMesh(core_axis_name="c", subcore_axis_name="s")

    @functools.partial(
        pl.kernel,
        out_type=[
            jax.ShapeDtypeStruct((2, N, D), jnp.float32),
            jax.ShapeDtypeStruct((2, NPAD), jnp.float32),
        ],
        mesh=mesh,
        compiler_params=pltpu.CompilerParams(
            needs_layout_passes=False, use_tc_tiling_on_sc=False),
        scratch_types=[
            pltpu.VMEM((3, CH), jnp.int32),
            pltpu.VMEM((3, CH), jnp.int32),
            pltpu.VMEM((3, CH), jnp.int32),
            pltpu.VMEM((3, 2 * CH), jnp.int32),
            pltpu.VMEM((3, 2 * CH), jnp.float32),
            pltpu.VMEM((3, CH), jnp.float32),
            pltpu.VMEM((3, CH, D), jnp.float32),
            pltpu.VMEM((RPT := NPAD // 16,), jnp.float32),
            pltpu.VMEM_SHARED((16, EPT), jnp.int32),
            pltpu.VMEM_SHARED((2 * NPAD,), jnp.float32),
            pltpu.VMEM_SHARED((N, D), jnp.float32),
            pltpu.VMEM_SHARED((NPAD,), jnp.float32),
            pltpu.SemaphoreType.DMA,
            pltpu.SemaphoreType.DMA,
            pltpu.SemaphoreType.DMA,
            pltpu.SemaphoreType.DMA,
            pltpu.SemaphoreType.DMA,
            pltpu.SemaphoreType.DMA,
            pltpu.SemaphoreType.DMA,
            pltpu.SemaphoreType.DMA,
            pltpu.SemaphoreType.DMA,
        ],
    )
    def edge_kernel(h_hbm, es_hbm, ed_hbm, pk_hbm,
                    num_out, den_out,
                    pk_v, src_v, dst_v, cidx_v, esedb_v, w_v, rows_v, zden_v,
                    edge_sh, esed_sh, num_acc, den_acc,
                    es0, es1, es2, gs0, gs1, gs2, ss0, ss1, ss2):
        cid = lax.axis_index("c")
        sid = lax.axis_index("s")
        wid = sid * 2 + cid
        esems = (es0, es1, es2)
        gsems = (gs0, gs1, gs2)
        ssems = (ss0, ss1, ss2)
        zero16 = jnp.zeros((16,), jnp.float32)
        RPTC = NPAD // 16

        def zrow(i, _):
            for c8 in range(D // 16):
                rows_v[0, i, pl.ds(c8 * 16, 16)] = zero16
            return 0
        lax.fori_loop(0, CH, zrow, 0)

        def zden(i, _):
            zden_v[pl.ds(i * 16, 16)] = zero16
            return 0
        lax.fori_loop(0, RPTC // 16, zden, 0)

        base_n = sid * NPT
        base_r = sid * RPTC
        for r in range(7):
            pltpu.sync_copy(rows_v.at[0],
                            num_acc.at[pl.ds(base_n + r * CH, CH)])
        pltpu.sync_copy(rows_v.at[0, pl.ds(0, NPT - 7 * CH)],
                        num_acc.at[pl.ds(base_n + 7 * CH, NPT - 7 * CH)])
        pltpu.sync_copy(zden_v, den_acc.at[pl.ds(base_r, RPTC)])

        pltpu.sync_copy(es_hbm.at[pl.ds(base_r, RPTC)],
                        esed_sh.at[pl.ds(base_r, RPTC)])
        pltpu.sync_copy(ed_hbm.at[pl.ds(base_r, RPTC)],
                        esed_sh.at[pl.ds(NPAD + base_r, RPTC)])
        pltpu.sync_copy(pk_hbm.at[pl.ds(wid * EPT, EPT)], edge_sh.at[sid])
        plsc.subcore_barrier()

        def prep(t, b):
            pltpu.sync_copy(edge_sh.at[sid, pl.ds(t * CH, CH)], pk_v.at[b])
            for g in range(G):
                sl = pl.ds(g * 16, 16)
                p16 = pk_v[b, sl]
                s16 = jnp.bitwise_and(p16, 16383)
                d16 = jnp.right_shift(p16, 14)
                src_v[b, sl] = s16
                dst_v[b, sl] = d16
                cidx_v[b, sl] = s16
                cidx_v[b, pl.ds(CH + g * 16, 16)] = d16 + NPAD
            pltpu.async_copy(esed_sh.at[cidx_v.at[b]], esedb_v.at[b],
                             esems[b])
            pltpu.async_copy(h_hbm.at[src_v.at[b]], rows_v.at[b], gsems[b])

        def wait_esed(b):
            pltpu.make_async_copy(esed_sh.at[cidx_v.at[b]], esedb_v.at[b],
                                  esems[b]).wait()

        def wait_gather(b):
            pltpu.make_async_copy(h_hbm.at[src_v.at[b]], rows_v.at[b],
                                  gsems[b]).wait()

        def start_scatter(b):
            pltpu.async_copy(rows_v.at[b], num_acc.at[dst_v.at[b]],
                             ssems[b], add=True)
            pltpu.async_copy(w_v.at[b], den_acc.at[dst_v.at[b]],
                             ssems[b], add=True)

        def wait_scatter(b):
            pltpu.make_async_copy(rows_v.at[b], num_acc.at[dst_v.at[b]],
                                  ssems[b]).wait()
            pltpu.make_async_copy(w_v.at[b], den_acc.at[dst_v.at[b]],
                                  ssems[b]).wait()

        def compute_w(b):
            wait_esed(b)
            for g in range(G):
                sl = pl.ds(g * 16, 16)
                e = esedb_v[b, sl] + esedb_v[b, pl.ds(CH + g * 16, 16)]
                e = jnp.where(e > 0, e, e * jnp.float32(0.2))
                e = jnp.minimum(e, jnp.float32(70.0))
                w_v[b, sl] = jnp.exp(e)

        def scale(b):
            def srow16(g, _):
                w16 = w_v[b, pl.ds(g * 16, 16)]
                for j in range(16):
                    wb = jnp.broadcast_to(w16[j], (16,))
                    i = g * 16 + j
                    for c8 in range(D // 16):
                        sl = pl.ds(c8 * 16, 16)
                        rows_v[b, i, sl] = rows_v[b, i, sl] * wb
                return 0
            lax.fori_loop(0, G, srow16, 0)

        def body(t, b):
            compute_w(b)
            b1 = (b + 1) % 3

            @pl.when(t >= 2)
            def _():
                wait_scatter(b1)

            @pl.when(t <= NCH - 2)
            def _():
                prep(t + 1, b1)
            wait_gather(b)
            scale(b)
            start_scatter(b)

        prep(0, 0)

        def uloop(u, _):
            for k in range(3):
                t = 3 * u + k

                @pl.when(t <= NCH - 1)
                def _():
                    body(t, k)
            return 0
        lax.fori_loop(0, (NCH + 2) // 3, uloop, 0)
        wait_scatter((NCH - 2) % 3)
        wait_scatter((NCH - 1) % 3)

        plsc.subcore_barrier()
        pltpu.sync_copy(num_acc.at[pl.ds(base_n, NPT)],
                        num_out.at[cid, pl.ds(base_n, NPT)])
        pltpu.sync_copy(den_acc.at[pl.ds(base_r, RPTC)],
                        den_out.at[cid, pl.ds(base_r, RPTC)])

    return edge_kernel


_edge128 = _make_edge_kernel(HID)


def kernel(x, edge_index, W1, a_src1, a_dst1, W2, a_src2, a_dst2,
           W3, a_src3, a_dst3):
    ei = edge_index.astype(jnp.int32)
    pk = jnp.bitwise_or(ei[0], jnp.left_shift(ei[1], 14))

    W3p = jnp.zeros((HID, OUT_PAD), jnp.float32).at[:, :40].set(W3)
    a_s3 = jnp.zeros((OUT_PAD,), jnp.float32).at[:40].set(a_src3)
    a_d3 = jnp.zeros((OUT_PAD,), jnp.float32).at[:40].set(a_dst3)

    h1, es1, ed1 = _prep1(x, W1, a_src1.reshape(HID, 1), a_dst1.reshape(HID, 1))
    num1, den1 = _edge128(h1, es1.reshape(NPAD), ed1.reshape(NPAD), pk)

    h2, es2, ed2 = _prep2(num1[0], num1[1],
                          den1[0].reshape(NPAD, 1), den1[1].reshape(NPAD, 1),
                          W2, a_src2.reshape(HID, 1), a_dst2.reshape(HID, 1))
    num2, den2 = _edge128(h2, es2.reshape(NPAD), ed2.reshape(NPAD), pk)

    h3, es3, ed3 = _prep3(num2[0], num2[1],
                          den2[0].reshape(NPAD, 1), den2[1].reshape(NPAD, 1),
                          W3p, a_s3.reshape(OUT_PAD, 1), a_d3.reshape(OUT_PAD, 1))
    num3, den3 = _edge128(h3, es3.reshape(NPAD), ed3.reshape(NPAD), pk)

    out = _final(num3[0], num3[1],
                 den3[0].reshape(NPAD, 1), den3[1].reshape(NPAD, 1))
    return out[:N, :40]
